# baseline (device time: 9706 ns/iter reference)
import jax
import jax.numpy as jnp
from jax import lax
from jax.experimental import pallas as pl
from jax.experimental.pallas import tpu as pltpu

N_DEV = 16


def kernel(x):
    m_per, n = x.shape

    def body(x_ref, out_ref, send_ref, recv_ref, send_sems, recv_sems):
        my = lax.axis_index("i")

        barrier_sem = pltpu.get_barrier_semaphore()
        for e in range(1, N_DEV):
            pl.semaphore_signal(
                barrier_sem, inc=1,
                device_id=((my + e) % N_DEV,),
                device_id_type=pl.DeviceIdType.MESH,
            )

        xv = x_ref[:, :]
        val = jnp.max(xv, axis=0)
        row_ids = lax.broadcasted_iota(jnp.int32, (m_per, n), 0)
        loc_idx = jnp.min(
            jnp.where(xv == val[None, :], row_ids, m_per), axis=0
        )
        best_val = val
        best_idx = (loc_idx + my * m_per).astype(jnp.float32)
        send_ref[0, :] = best_val
        send_ref[1, :] = best_idx
        recv_ref[0, 0, :] = best_val
        recv_ref[0, 1, :] = best_idx

        pl.semaphore_wait(barrier_sem, N_DEV - 1)

        rdmas = []
        for e in range(1, N_DEV):
            rdma = pltpu.make_async_remote_copy(
                src_ref=send_ref,
                dst_ref=recv_ref.at[e],
                send_sem=send_sems.at[e],
                recv_sem=recv_sems.at[e],
                device_id=((my + e) % N_DEV,),
                device_id_type=pl.DeviceIdType.MESH,
            )
            rdma.start()
            rdmas.append(rdma)

        for e in range(1, N_DEV):
            rdmas[e - 1].wait_recv()

        vals = recv_ref[:, 0, :]
        idxs = recv_ref[:, 1, :]
        gmax = jnp.max(vals, axis=0)
        gidx = jnp.min(
            jnp.where(vals == gmax[None, :], idxs, jnp.float32(1e9)), axis=0
        )
        out_ref[0, :] = gmax
        out_ref[1, :] = gidx

        for r in rdmas:
            r.wait_send()

    return pl.pallas_call(
        body,
        out_shape=jax.ShapeDtypeStruct((2, n), jnp.float32),
        in_specs=[pl.BlockSpec(memory_space=pltpu.VMEM)],
        out_specs=pl.BlockSpec(memory_space=pltpu.VMEM),
        scratch_shapes=[
            pltpu.VMEM((2, n), jnp.float32),
            pltpu.VMEM((N_DEV, 2, n), jnp.float32),
            pltpu.SemaphoreType.DMA((N_DEV,)),
            pltpu.SemaphoreType.DMA((N_DEV,)),
        ],
        compiler_params=pltpu.CompilerParams(collective_id=0),
    )(x)
